# Initial kernel scaffold; baseline (speedup 1.0000x reference)
import functools

import jax
import jax.numpy as jnp
from jax import lax
from jax.experimental import pallas as pl
from jax.experimental.pallas import tpu as pltpu
from jax.experimental.pallas import tpu_sc as plsc

VOCAB = 1000
ROWS = 1024 * 20
NC, NS = 2, 16
NW = NC * NS
B_PER_W = ROWS // NW
CHUNK = 40
NCHUNK = B_PER_W // CHUNK


def _sc_gather(table, flat_idx):
    mesh = plsc.VectorSubcoreMesh(core_axis_name="c", subcore_axis_name="s")

    @functools.partial(
        pl.kernel,
        mesh=mesh,
        out_type=jax.ShapeDtypeStruct((ROWS, VOCAB), jnp.float32),
        scratch_types=[
            pltpu.VMEM((B_PER_W,), jnp.int32),
            pltpu.VMEM((CHUNK, VOCAB), jnp.float32),
            pltpu.SemaphoreType.DMA,
        ],
    )
    def k(table_hbm, idx_hbm, out_hbm, idx_v, buf0, gsem):
        wid = lax.axis_index("s") * NC + lax.axis_index("c")
        base = wid * B_PER_W
        pltpu.sync_copy(idx_hbm.at[pl.ds(base, B_PER_W)], idx_v)

        def body(i, _):
            pltpu.async_copy(
                table_hbm.at[idx_v.at[pl.ds(i * CHUNK, CHUNK)]], buf0, gsem
            ).wait()
            pltpu.sync_copy(buf0, out_hbm.at[pl.ds(base + i * CHUNK, CHUNK)])
            return 0

        lax.fori_loop(0, NCHUNK, body, 0)

    return k(table, flat_idx)


def kernel(inputs, table):
    B, L = inputs.shape
    flat_idx = inputs.reshape(-1).astype(jnp.int32)
    out = _sc_gather(table, flat_idx)
    return out.reshape(B, L * VOCAB)


# speedup vs baseline: 1.4106x; 1.4106x over previous
import functools

import jax
import jax.numpy as jnp
from jax import lax
from jax.experimental import pallas as pl
from jax.experimental.pallas import tpu as pltpu
from jax.experimental.pallas import tpu_sc as plsc

VOCAB = 1000
ROWS = 1024 * 20
NC, NS = 2, 16
NW = NC * NS
B_PER_W = ROWS // NW
CHUNK = 40
NCHUNK = B_PER_W // CHUNK


def _sc_gather(table, flat_idx):
    mesh = plsc.VectorSubcoreMesh(core_axis_name="c", subcore_axis_name="s")

    @functools.partial(
        pl.kernel,
        mesh=mesh,
        out_type=jax.ShapeDtypeStruct((ROWS, VOCAB), jnp.float32),
        scratch_types=[
            pltpu.VMEM((B_PER_W,), jnp.int32),
            pltpu.VMEM((CHUNK, VOCAB), jnp.float32),
            pltpu.SemaphoreType.DMA,
        ],
        compiler_params=pltpu.CompilerParams(use_tc_tiling_on_sc=False),
    )
    def k(table_hbm, idx_hbm, out_hbm, idx_v, buf0, gsem):
        wid = lax.axis_index("s") * NC + lax.axis_index("c")
        base = wid * B_PER_W
        pltpu.sync_copy(idx_hbm.at[pl.ds(base, B_PER_W)], idx_v)

        def body(i, _):
            pltpu.async_copy(
                table_hbm.at[idx_v.at[pl.ds(i * CHUNK, CHUNK)]], buf0, gsem
            ).wait()
            pltpu.sync_copy(buf0, out_hbm.at[pl.ds(base + i * CHUNK, CHUNK)])
            return 0

        lax.fori_loop(0, NCHUNK, body, 0)

    return k(table, flat_idx)


def kernel(inputs, table):
    B, L = inputs.shape
    flat_idx = inputs.reshape(-1).astype(jnp.int32)
    out = _sc_gather(table, flat_idx)
    return out.reshape(B, L * VOCAB)


# speedup vs baseline: 1.4736x; 1.0447x over previous
import functools

import jax
import jax.numpy as jnp
from jax import lax
from jax.experimental import pallas as pl
from jax.experimental.pallas import tpu as pltpu
from jax.experimental.pallas import tpu_sc as plsc

VOCAB = 1000
ROWS = 1024 * 20
NC, NS = 2, 16
NW = NC * NS
B_PER_W = ROWS // NW
CHUNK = 64
NCHUNK = B_PER_W // CHUNK


def _sc_gather(table, flat_idx):
    mesh = plsc.VectorSubcoreMesh(core_axis_name="c", subcore_axis_name="s")

    @functools.partial(
        pl.kernel,
        mesh=mesh,
        out_type=jax.ShapeDtypeStruct((ROWS, VOCAB), jnp.float32),
        scratch_types=[
            pltpu.VMEM((B_PER_W,), jnp.int32),
            pltpu.VMEM((CHUNK, VOCAB), jnp.float32),
            pltpu.VMEM((CHUNK, VOCAB), jnp.float32),
            pltpu.SemaphoreType.DMA,
            pltpu.SemaphoreType.DMA,
            pltpu.SemaphoreType.DMA,
            pltpu.SemaphoreType.DMA,
        ],
        compiler_params=pltpu.CompilerParams(use_tc_tiling_on_sc=False),
    )
    def k(table_hbm, idx_hbm, out_hbm, idx_v, buf0, buf1, g0, g1, s0, s1):
        wid = lax.axis_index("s") * NC + lax.axis_index("c")
        base = wid * B_PER_W
        pltpu.sync_copy(idx_hbm.at[pl.ds(base, B_PER_W)], idx_v)

        bufs = (buf0, buf1)
        gsems = (g0, g1)
        ssems = (s0, s1)

        def gather(i):
            return pltpu.async_copy(
                table_hbm.at[idx_v.at[pl.ds(i * CHUNK, CHUNK)]],
                bufs[i % 2],
                gsems[i % 2],
            )

        def store(i):
            return pltpu.async_copy(
                bufs[i % 2],
                out_hbm.at[pl.ds(base + i * CHUNK, CHUNK)],
                ssems[i % 2],
            )

        gd = [None] * NCHUNK
        sd = [None] * NCHUNK
        gd[0] = gather(0)
        gd[1] = gather(1)
        gd[0].wait()
        sd[0] = store(0)
        for i in range(1, NCHUNK):
            sd[i - 1].wait()
            if i + 1 < NCHUNK:
                gd[i + 1] = gather(i + 1)
            gd[i].wait()
            sd[i] = store(i)
        sd[NCHUNK - 1].wait()

    return k(table, flat_idx)


def kernel(inputs, table):
    B, L = inputs.shape
    flat_idx = inputs.reshape(-1).astype(jnp.int32)
    out = _sc_gather(table, flat_idx)
    return out.reshape(B, L * VOCAB)
